# baseline (speedup 1.0000x reference)
import functools

import jax
import jax.numpy as jnp
from jax import lax
from jax.experimental import pallas as pl
from jax.experimental.pallas import tpu as pltpu
from jax.experimental.pallas import tpu_sc as plsc

VOCAB = 10
HIDDEN = 4096
PRE = 16
BATCH = 4
SEQ = 8192

NC = 2
NS = 16
NW = NC * NS
ROWS_PER_W = BATCH * SEQ // NW
GR = 8
K = 4
GK = GR // K
NG = ROWS_PER_W // GR
WPB = NW // BATCH


def _sc_embed(ids3, word_embeddings, prompt_embeddings):
    mesh = plsc.VectorSubcoreMesh(core_axis_name="c", subcore_axis_name="s")

    @functools.partial(
        pl.kernel,
        mesh=mesh,
        out_type=jax.ShapeDtypeStruct((BATCH, PRE + SEQ, HIDDEN), jnp.float32),
        scratch_types=(
            [pltpu.VMEM((NG * K, GK), jnp.int32)]
            + [pltpu.VMEM((GK, HIDDEN), jnp.float32) for _ in range(2 * K)]
            + [pltpu.SemaphoreType.DMA for _ in range(4)]
        ),
    )
    def k(ids_hbm, we_hbm, pe_hbm, out_hbm, idx_v, *rest):
        bufs = (rest[0:K], rest[K:2 * K])
        sgs = rest[2 * K:2 * K + 2]
        sws = rest[2 * K + 2:2 * K + 4]
        wid = lax.axis_index("s") * NC + lax.axis_index("c")
        b = wid // WPB
        s0 = (wid % WPB) * ROWS_PER_W
        pltpu.sync_copy(ids_hbm.at[wid], idx_v)

        def gather(r, pg, kk):
            return pltpu.make_async_copy(
                we_hbm.at[idx_v.at[r * K + kk]], bufs[pg][kk], sgs[pg]
            )

        def write(r, pg, kk):
            return pltpu.make_async_copy(
                bufs[pg][kk],
                out_hbm.at[b, pl.ds(PRE + s0 + r * GR + kk * GK, GK)],
                sws[pg],
            )

        for pg in range(2):
            for kk in range(K):
                gather(pg, pg, kk).start()

        def body(r2, carry):
            for pg in range(2):
                r = 2 * r2 + pg
                for kk in range(K):
                    gather(r, pg, kk).wait()
                for kk in range(K):
                    write(r, pg, kk).start()
                for kk in range(K):
                    write(r, pg, kk).wait()

                @pl.when(r + 2 < NG)
                def _():
                    for kk in range(K):
                        gather(r + 2, pg, kk).start()

            return carry

        lax.fori_loop(0, NG // 2, body, 0)

        @pl.when(wid % WPB == 0)
        def _():
            for q in range(PRE // GK):
                pltpu.sync_copy(pe_hbm.at[pl.ds(q * GK, GK)], bufs[0][0])
                pltpu.sync_copy(bufs[0][0], out_hbm.at[b, pl.ds(q * GK, GK)])

    return k(ids3, word_embeddings, prompt_embeddings)


@jax.jit
def kernel(input_ids, word_embeddings, prompt_embeddings):
    ids3 = input_ids.astype(jnp.int32).reshape(NW, NG * K, GK)
    return _sc_embed(ids3, word_embeddings, prompt_embeddings)
